# baseline (speedup 1.0000x reference)
import jax
import jax.numpy as jnp
from jax import lax
from jax.experimental import pallas as pl
from jax.experimental.pallas import tpu as pltpu
from jax.experimental.pallas import tpu_sc as plsc

N_CH = 192
N_PTS = 7
BOUND = 2.7
N_REG = N_PTS - 1
ROW = 224 * 224
NROWS = 4 * N_CH
NW = 32
NBUF = 4
CPR = 4
CHUNK = ROW // CPR
LANES = 16

_INV_LEN = float(N_REG) / (2.0 * BOUND)
_SHIFT = BOUND * _INV_LEN


def _take16(vec, idx):
  return vec.at[idx].get(mode="promise_in_bounds")


def _body(x_hbm, pts_hbm, out_hbm, pts_row, inbufs, outbufs, isems, osems):
  wid = lax.axis_index("s") * 2 + lax.axis_index("c")
  rows_per_w = NROWS // NW
  chunks_per_w = rows_per_w * CPR
  base_chunk = wid * chunks_per_w

  def start_in(g, b):
    pltpu.async_copy(x_hbm.at[pl.ds((base_chunk + g) * CHUNK, CHUNK)],
                     inbufs[b], isems[b])

  def wait_in(b):
    pltpu.make_async_copy(x_hbm.at[pl.ds(0, CHUNK)], inbufs[b],
                          isems[b]).wait()

  def start_out(g, b):
    pltpu.async_copy(outbufs[b],
                     out_hbm.at[pl.ds((base_chunk + g) * CHUNK, CHUNK)],
                     osems[b])

  def wait_out(b):
    pltpu.make_async_copy(outbufs[b], out_hbm.at[pl.ds(0, CHUNK)],
                          osems[b]).wait()

  for b in range(NBUF):
    start_in(b, b)

  lanes = lax.iota(jnp.int32, LANES)
  shift_idx = jnp.minimum(lanes + 1, LANES - 1)
  lanes_f = lanes.astype(jnp.float32)

  def row_body(j, carry):
    row = wid * rows_per_w + j
    ch = lax.rem(row, N_CH)
    pltpu.sync_copy(pts_hbm.at[ch], pts_row)
    p = pts_row[...]
    d = _take16(p, shift_idx) - p
    a = p - lanes_f * d

    for bb in range(CPR):
      g = j * CPR + bb
      b = bb
      if True:
        wait_in(b)

        @pl.when(g >= NBUF)
        def _():
          wait_out(b)

        @plsc.parallel_loop(0, CHUNK, step=LANES, unroll=32)
        def _(off):
          xv = inbufs[b][pl.ds(off, LANES)]
          xn = xv * _INV_LEN + _SHIFT
          ri = jnp.clip(xn, 0.0, 5.9999995).astype(jnp.int32)
          outbufs[b][pl.ds(off, LANES)] = (
              _take16(a, ri) + xn * _take16(d, ri))

        start_out(g, b)

        @pl.when(g < chunks_per_w - NBUF)
        def _():
          start_in(g + NBUF, b)

    return carry

  lax.fori_loop(0, rows_per_w, row_body, 0)
  for b in range(NBUF):
    wait_out(b)


@jax.jit
def _pwlu_sc(x_flat, pts_pad):
  mesh = plsc.VectorSubcoreMesh(core_axis_name="c", subcore_axis_name="s")
  return pl.kernel(
      _body,
      out_type=jax.ShapeDtypeStruct((NROWS * ROW,), jnp.float32),
      mesh=mesh,
      scratch_types=[
          pltpu.VMEM((LANES,), jnp.float32),
          [pltpu.VMEM((CHUNK,), jnp.float32) for _ in range(NBUF)],
          [pltpu.VMEM((CHUNK,), jnp.float32) for _ in range(NBUF)],
          [pltpu.SemaphoreType.DMA for _ in range(NBUF)],
          [pltpu.SemaphoreType.DMA for _ in range(NBUF)],
      ],
  )(x_flat, pts_pad)


def kernel(x, points):
  pts_pad = jnp.zeros((N_CH, LANES), jnp.float32).at[:, :N_PTS].set(points)
  out = _pwlu_sc(x.reshape(-1), pts_pad)
  return out.reshape(x.shape)


# speedup vs baseline: 1.3107x; 1.3107x over previous
import jax
import jax.numpy as jnp
from jax import lax
from jax.experimental import pallas as pl
from jax.experimental.pallas import tpu as pltpu
from jax.experimental.pallas import tpu_sc as plsc

N_CH = 192
N_PTS = 7
BOUND = 2.7
N_REG = N_PTS - 1
ROW = 224 * 224
NROWS = 4 * N_CH
NW = 32
NBUF = 4
CPR = 4
CHUNK = ROW // CPR
LANES = 16

_INV_LEN = float(N_REG) / (2.0 * BOUND)
_SHIFT = BOUND * _INV_LEN


def _take16(vec, idx):
  return vec.at[idx].get(mode="promise_in_bounds")


def _body(x_hbm, pts_hbm, out_hbm, pts_row, inbufs, outbufs, isems, osems):
  wid = lax.axis_index("s") * 2 + lax.axis_index("c")
  rows_per_w = NROWS // NW
  chunks_per_w = rows_per_w * CPR
  base_chunk = wid * chunks_per_w

  def start_in(g, b):
    pltpu.async_copy(x_hbm.at[pl.ds((base_chunk + g) * CHUNK, CHUNK)],
                     inbufs[b], isems[b])

  def wait_in(b):
    pltpu.make_async_copy(x_hbm.at[pl.ds(0, CHUNK)], inbufs[b],
                          isems[b]).wait()

  def start_out(g, b):
    pltpu.async_copy(outbufs[b],
                     out_hbm.at[pl.ds((base_chunk + g) * CHUNK, CHUNK)],
                     osems[b])

  def wait_out(b):
    pltpu.make_async_copy(outbufs[b], out_hbm.at[pl.ds(0, CHUNK)],
                          osems[b]).wait()

  for b in range(NBUF):
    start_in(b, b)

  lanes = lax.iota(jnp.int32, LANES)
  shift_idx = jnp.minimum(lanes + 1, LANES - 1)
  lanes_f = lanes.astype(jnp.float32)

  def row_body(j, carry):
    row = wid * rows_per_w + j
    ch = lax.rem(row, N_CH)
    pltpu.sync_copy(pts_hbm.at[ch], pts_row)
    p = pts_row[...]
    d = _take16(p, shift_idx) - p
    a = p - lanes_f * d

    for bb in range(CPR):
      g = j * CPR + bb
      b = bb
      if True:
        wait_in(b)

        @pl.when(g >= NBUF)
        def _():
          wait_out(b)

        @plsc.parallel_loop(0, CHUNK, step=LANES, unroll=16)
        def _(off):
          xv = inbufs[b][pl.ds(off, LANES)]
          xn = xv * _INV_LEN + _SHIFT
          ri = jnp.clip(xn, 0.0, 5.9999995).astype(jnp.int32)
          outbufs[b][pl.ds(off, LANES)] = (
              _take16(a, ri) + xn * _take16(d, ri))

        start_out(g, b)

        @pl.when(g < chunks_per_w - NBUF)
        def _():
          start_in(g + NBUF, b)

    return carry

  lax.fori_loop(0, rows_per_w, row_body, 0)
  for b in range(NBUF):
    wait_out(b)


@jax.jit
def _pwlu_sc(x_flat, pts_pad):
  mesh = plsc.VectorSubcoreMesh(core_axis_name="c", subcore_axis_name="s")
  return pl.kernel(
      _body,
      out_type=jax.ShapeDtypeStruct((NROWS * ROW,), jnp.float32),
      mesh=mesh,
      scratch_types=[
          pltpu.VMEM((LANES,), jnp.float32),
          [pltpu.VMEM((CHUNK,), jnp.float32) for _ in range(NBUF)],
          [pltpu.VMEM((CHUNK,), jnp.float32) for _ in range(NBUF)],
          [pltpu.SemaphoreType.DMA for _ in range(NBUF)],
          [pltpu.SemaphoreType.DMA for _ in range(NBUF)],
      ],
  )(x_flat, pts_pad)


def kernel(x, points):
  pts_pad = jnp.zeros((N_CH, LANES), jnp.float32).at[:, :N_PTS].set(points)
  out = _pwlu_sc(x.reshape(-1), pts_pad)
  return out.reshape(x.shape)


# speedup vs baseline: 1.3119x; 1.0009x over previous
import jax
import jax.numpy as jnp
from jax import lax
from jax.experimental import pallas as pl
from jax.experimental.pallas import tpu as pltpu
from jax.experimental.pallas import tpu_sc as plsc

N_CH = 192
N_PTS = 7
BOUND = 2.7
N_REG = N_PTS - 1
ROW = 224 * 224
NROWS = 4 * N_CH
NW = 32
NBUF = 4
CPR = 4
CHUNK = ROW // CPR
LANES = 16

_INV_LEN = float(N_REG) / (2.0 * BOUND)
_SHIFT = BOUND * _INV_LEN


def _take16(vec, idx):
  return vec.at[idx].get(mode="promise_in_bounds")


def _body(x_hbm, pts_hbm, out_hbm, pts_row, inbufs, outbufs, isems, osems):
  wid = lax.axis_index("s") * 2 + lax.axis_index("c")
  rows_per_w = NROWS // NW
  chunks_per_w = rows_per_w * CPR
  base_chunk = wid * chunks_per_w

  def start_in(g, b):
    pltpu.async_copy(x_hbm.at[pl.ds((base_chunk + g) * CHUNK, CHUNK)],
                     inbufs[b], isems[b])

  def wait_in(b):
    pltpu.make_async_copy(x_hbm.at[pl.ds(0, CHUNK)], inbufs[b],
                          isems[b]).wait()

  def start_out(g, b):
    pltpu.async_copy(outbufs[b],
                     out_hbm.at[pl.ds((base_chunk + g) * CHUNK, CHUNK)],
                     osems[b])

  def wait_out(b):
    pltpu.make_async_copy(outbufs[b], out_hbm.at[pl.ds(0, CHUNK)],
                          osems[b]).wait()

  for b in range(NBUF):
    start_in(b, b)

  lanes = lax.iota(jnp.int32, LANES)
  shift_idx = jnp.minimum(lanes + 1, LANES - 1)
  lanes_f = lanes.astype(jnp.float32)

  def row_body(j, carry):
    row = wid * rows_per_w + j
    ch = lax.rem(row, N_CH)
    pltpu.sync_copy(pts_hbm.at[ch], pts_row)
    p = pts_row[...]
    d = _take16(p, shift_idx) - p
    a = p - lanes_f * d

    for bb in range(CPR):
      g = j * CPR + bb
      b = bb
      wait_in(b)

      @pl.when(g >= NBUF)
      def _():
        wait_out(b)

      @plsc.parallel_loop(0, CHUNK, step=LANES, unroll=16)
      def _(off):
        xv = inbufs[b][pl.ds(off, LANES)]
        xn = xv * _INV_LEN + _SHIFT
        ri = jnp.clip(xn, 0.0, 5.9999995).astype(jnp.int32)
        outbufs[b][pl.ds(off, LANES)] = (
            _take16(a, ri) + xn * _take16(d, ri))

      start_out(g, b)

      @pl.when(g < chunks_per_w - NBUF)
      def _():
        start_in(g + NBUF, b)

    return carry

  lax.fori_loop(0, rows_per_w, row_body, 0)
  for b in range(NBUF):
    wait_out(b)


@jax.jit
def _pwlu_sc(x_flat, pts_pad):
  mesh = plsc.VectorSubcoreMesh(core_axis_name="c", subcore_axis_name="s")
  return pl.kernel(
      _body,
      out_type=jax.ShapeDtypeStruct((NROWS * ROW,), jnp.float32),
      mesh=mesh,
      scratch_types=[
          pltpu.VMEM((LANES,), jnp.float32),
          [pltpu.VMEM((CHUNK,), jnp.float32) for _ in range(NBUF)],
          [pltpu.VMEM((CHUNK,), jnp.float32) for _ in range(NBUF)],
          [pltpu.SemaphoreType.DMA for _ in range(NBUF)],
          [pltpu.SemaphoreType.DMA for _ in range(NBUF)],
      ],
  )(x_flat, pts_pad)


def kernel(x, points):
  pts_pad = jnp.zeros((N_CH, LANES), jnp.float32).at[:, :N_PTS].set(points)
  out = _pwlu_sc(x.reshape(-1), pts_pad)
  return out.reshape(x.shape)
